# baseline (speedup 1.0000x reference)
import functools

import jax
import jax.numpy as jnp
from jax import lax
from jax.experimental import pallas as pl
from jax.experimental.pallas import tpu as pltpu
from jax.experimental.pallas import tpu_sc as plsc

N = 10000
D = 128
H = 128
C = 121

NC = 2
NS = 16
NW = NC * NS
LANES = 16

NP = 10112
RPT = NP // NS
CH = 112
E_TOT = 320000 + N
NCHUNK = -(-E_TOT // (NW * CH))
EPT = NCHUNK * CH
E_PAD = EPT * NW

@functools.cache
def _sc_kernels():
    mesh = plsc.VectorSubcoreMesh(core_axis_name="c", subcore_axis_name="s",
                                  num_cores=NC, num_subcores=NS)

    @functools.partial(
        pl.kernel,
        out_type=jax.ShapeDtypeStruct((NC, NP, LANES), jnp.float32),
        mesh=mesh,
        scratch_types=[
            pltpu.VMEM((NCHUNK, 2, CH), jnp.int32),
            pltpu.VMEM((CH, LANES), jnp.float32),
            pltpu.VMEM_SHARED((NP, LANES), jnp.float32),
            pltpu.SemaphoreType.DMA,
        ],
    )
    def _deg(edges_hbm, zeros_hbm, ones_hbm, out_hbm, idx_all, ones_v, acc, sem):
        c = lax.axis_index("c")
        s = lax.axis_index("s")
        wid = s * NC + c
        rbase = s * RPT
        pltpu.sync_copy(ones_hbm, ones_v)
        pltpu.sync_copy(edges_hbm.at[wid], idx_all)
        pltpu.sync_copy(zeros_hbm.at[pl.ds(rbase, RPT)], acc.at[pl.ds(rbase, RPT)])
        plsc.subcore_barrier()

        def body(j, carry):
            pltpu.sync_copy(ones_v, acc.at[idx_all.at[j, 1]], add=True)
            return carry

        lax.fori_loop(0, NCHUNK, body, 0)
        plsc.subcore_barrier()
        pltpu.sync_copy(acc.at[pl.ds(rbase, RPT)], out_hbm.at[c, pl.ds(rbase, RPT)])

    NB = 3
    NIB = 4

    @functools.partial(
        pl.kernel,
        out_type=jax.ShapeDtypeStruct((NC, NP, H), jnp.float32),
        mesh=mesh,
        scratch_types=[
            pltpu.VMEM((NIB, 2, CH), jnp.int32),
            pltpu.VMEM((NB, CH, H), jnp.float32),
            pltpu.VMEM_SHARED((NP, H), jnp.float32),
            pltpu.SemaphoreType.DMA,
            pltpu.SemaphoreType.DMA,
            pltpu.SemaphoreType.DMA,
        ],
    )
    def _agg(g_hbm, edges_hbm, zeros_hbm, out_hbm,
             idx_v, rows_v, acc, semi, semg, sems):
        c = lax.axis_index("c")
        s = lax.axis_index("s")
        wid = s * NC + c
        rbase = s * RPT
        pltpu.sync_copy(zeros_hbm.at[pl.ds(rbase, RPT)], acc.at[pl.ds(rbase, RPT)])
        for j0 in range(NIB):
            pltpu.async_copy(edges_hbm.at[wid, j0], idx_v.at[j0], semi)
        plsc.subcore_barrier()
        for j0 in range(NB - 1):
            pltpu.make_async_copy(edges_hbm.at[wid, j0], idx_v.at[j0], semi).wait()
            pltpu.async_copy(g_hbm.at[idx_v.at[j0, 0]], rows_v.at[j0], semg)

        def body(j, carry):
            b = j % NB
            i = j % NIB
            pltpu.make_async_copy(g_hbm.at[pl.ds(0, CH)], rows_v.at[b], semg).wait()
            pltpu.async_copy(rows_v.at[b], acc.at[idx_v.at[i, 1]], sems, add=True)

            @pl.when(j > 0)
            def _():
                pltpu.make_async_copy(g_hbm.at[pl.ds(0, CH)], acc.at[pl.ds(0, CH)], sems).wait()

                @pl.when(j - 1 + NIB < NCHUNK)
                def _():
                    pltpu.async_copy(edges_hbm.at[wid, j - 1 + NIB], idx_v.at[(j - 1) % NIB], semi)

            @pl.when(j + NB - 1 < NCHUNK)
            def _():
                pltpu.make_async_copy(edges_hbm.at[wid, 0], idx_v.at[i], semi).wait()
                pltpu.async_copy(g_hbm.at[idx_v.at[(j + NB - 1) % NIB, 0]],
                                 rows_v.at[(j + NB - 1) % NB], semg)

            return carry

        lax.fori_loop(0, NCHUNK, body, 0)
        pltpu.make_async_copy(g_hbm.at[pl.ds(0, CH)], acc.at[pl.ds(0, CH)], sems).wait()
        plsc.subcore_barrier()
        pltpu.sync_copy(acc.at[pl.ds(rbase, RPT)], out_hbm.at[c, pl.ds(rbase, RPT)])

    return _deg, _agg


R = 1000
_GRID = N // R


def _tc0_body(deg_ref, x_ref, w_ref, dinv_ref, g_ref):
    deg = deg_ref[0, :, 0:1] + deg_ref[1, :, 0:1]
    dinv = lax.rsqrt(deg)
    dinv_ref[...] = dinv
    h = jnp.dot(x_ref[...], w_ref[...], preferred_element_type=jnp.float32)
    g_ref[...] = h * dinv


def _tc0(degp, x, w0):
    return pl.pallas_call(
        _tc0_body,
        grid=(_GRID,),
        in_specs=[
            pl.BlockSpec((NC, R, LANES), lambda i: (0, i, 0)),
            pl.BlockSpec((R, D), lambda i: (i, 0)),
            pl.BlockSpec((D, H), lambda i: (0, 0)),
        ],
        out_specs=[
            pl.BlockSpec((R, 1), lambda i: (i, 0)),
            pl.BlockSpec((R, H), lambda i: (i, 0)),
        ],
        out_shape=[
            jax.ShapeDtypeStruct((N, 1), jnp.float32),
            jax.ShapeDtypeStruct((N, H), jnp.float32),
        ],
    )(degp, x, w0)


def _tcmid_body(sp_ref, dinv_ref, b_ref, w_ref, g_ref):
    dinv = dinv_ref[...]
    ssum = sp_ref[0] + sp_ref[1]
    y = jnp.maximum(ssum * dinv + b_ref[...], 0.0)
    g_ref[...] = jnp.dot(y, w_ref[...], preferred_element_type=jnp.float32) * dinv


def _tcmid(sp, dinv, b, w):
    return pl.pallas_call(
        _tcmid_body,
        grid=(_GRID,),
        in_specs=[
            pl.BlockSpec((NC, R, H), lambda i: (0, i, 0)),
            pl.BlockSpec((R, 1), lambda i: (i, 0)),
            pl.BlockSpec((1, H), lambda i: (0, 0)),
            pl.BlockSpec((H, H), lambda i: (0, 0)),
        ],
        out_specs=pl.BlockSpec((R, H), lambda i: (i, 0)),
        out_shape=jax.ShapeDtypeStruct((N, H), jnp.float32),
    )(sp, dinv, b, w)


def _tcfinal_body(sp_ref, dinv_ref, b2_ref, w1_ref, b1_ref, w2_ref, b2f_ref, w3_ref, b3_ref, out_ref):
    dinv = dinv_ref[...]
    y = jnp.maximum((sp_ref[0] + sp_ref[1]) * dinv + b2_ref[...], 0.0)
    z = jnp.maximum(jnp.dot(y, w1_ref[...], preferred_element_type=jnp.float32) + b1_ref[...], 0.0)
    z = jnp.maximum(jnp.dot(z, w2_ref[...], preferred_element_type=jnp.float32) + b2f_ref[...], 0.0)
    out_ref[...] = jnp.dot(z, w3_ref[...], preferred_element_type=jnp.float32) + b3_ref[...]


def _tcfinal(sp, dinv, b2, fc1W, fc1b, fc2W, fc2b, fc3Wp, fc3bp):
    return pl.pallas_call(
        _tcfinal_body,
        grid=(_GRID,),
        in_specs=[
            pl.BlockSpec((NC, R, H), lambda i: (0, i, 0)),
            pl.BlockSpec((R, 1), lambda i: (i, 0)),
            pl.BlockSpec((1, H), lambda i: (0, 0)),
            pl.BlockSpec((H, H), lambda i: (0, 0)),
            pl.BlockSpec((1, H), lambda i: (0, 0)),
            pl.BlockSpec((H, H), lambda i: (0, 0)),
            pl.BlockSpec((1, H), lambda i: (0, 0)),
            pl.BlockSpec((H, 128), lambda i: (0, 0)),
            pl.BlockSpec((1, 128), lambda i: (0, 0)),
        ],
        out_specs=pl.BlockSpec((R, 128), lambda i: (i, 0)),
        out_shape=jax.ShapeDtypeStruct((N, 128), jnp.float32),
    )(sp, dinv, b2, fc1W, fc1b, fc2W, fc2b, fc3Wp, fc3bp)


def kernel(x, edge_index, convW0, convb0, convW1, convb1, convW2, convb2,
           fc1W, fc1b, fc2W, fc2b, fc3W, fc3b, TRAIN=False):
    del TRAIN
    loop = jnp.arange(N, dtype=jnp.int32)
    pad = E_PAD - E_TOT
    srcp = jnp.concatenate([edge_index[0], loop, jnp.zeros((pad,), jnp.int32)]).reshape(NW, NCHUNK, CH)
    dstp = jnp.concatenate([edge_index[1], loop, jnp.full((pad,), N, jnp.int32)]).reshape(NW, NCHUNK, CH)
    edges = jnp.stack([srcp, dstp], axis=2)

    zeros_w = jnp.zeros((NP, H), jnp.float32)
    zeros_l = jnp.zeros((NP, LANES), jnp.float32)
    ones_l = jnp.ones((CH, LANES), jnp.float32)

    _deg, _agg = _sc_kernels()
    degp = _deg(edges, zeros_l, ones_l)
    dinv, g = _tc0(degp, x, convW0)
    for (b_prev, w_next) in ((convb0, convW1), (convb1, convW2)):
        sp = _agg(g, edges, zeros_w)
        g = _tcmid(sp, dinv, b_prev.reshape(1, H), w_next)
    sp = _agg(g, edges, zeros_w)

    fc3Wp = jnp.pad(fc3W, ((0, 0), (0, 128 - C)))
    fc3bp = jnp.pad(fc3b, (0, 128 - C)).reshape(1, 128)
    out = _tcfinal(sp, dinv, convb2.reshape(1, H), fc1W, fc1b.reshape(1, H),
                   fc2W, fc2b.reshape(1, H), fc3Wp, fc3bp)
    return out[:, :C]


# speedup vs baseline: 1.2524x; 1.2524x over previous
import functools

import jax
import jax.numpy as jnp
from jax import lax
from jax.experimental import pallas as pl
from jax.experimental.pallas import tpu as pltpu
from jax.experimental.pallas import tpu_sc as plsc

N = 10000
D = 128
H = 128
C = 121

NC = 2
NS = 16
NW = NC * NS
LANES = 16

NP = 10112
RPT = NP // NS
CH = 128
E_TOT = 320000 + N
NCHUNK = -(-E_TOT // (NW * CH))
E_PAD = NCHUNK * CH * NW
K0 = 65
K1 = 2 * NCHUNK - K0
KMAX = max(K0, K1)

@functools.cache
def _sc_kernels():
    mesh = plsc.VectorSubcoreMesh(core_axis_name="c", subcore_axis_name="s",
                                  num_cores=NC, num_subcores=NS)

    @functools.partial(
        pl.kernel,
        out_type=jax.ShapeDtypeStruct((NC, NP, LANES), jnp.float32),
        mesh=mesh,
        scratch_types=[
            pltpu.VMEM((KMAX, 2, CH), jnp.int32),
            pltpu.VMEM((CH, LANES), jnp.float32),
            pltpu.VMEM_SHARED((NP, LANES), jnp.float32),
            pltpu.SemaphoreType.DMA,
        ],
    )
    def _deg(edges_hbm, zeros_hbm, ones_hbm, out_hbm, idx_all, ones_v, acc, sem):
        c = lax.axis_index("c")
        s = lax.axis_index("s")
        wid = s * NC + c
        kb = jnp.where(c == 0, K0, K1)
        rbase = s * RPT
        pltpu.sync_copy(ones_hbm, ones_v)
        pltpu.sync_copy(edges_hbm.at[wid], idx_all)
        pltpu.sync_copy(zeros_hbm.at[pl.ds(rbase, RPT)], acc.at[pl.ds(rbase, RPT)])
        plsc.subcore_barrier()

        def body(j, carry):
            pltpu.sync_copy(ones_v, acc.at[idx_all.at[j, 1]], add=True)
            return carry

        lax.fori_loop(0, kb, body, 0)
        plsc.subcore_barrier()
        pltpu.sync_copy(acc.at[pl.ds(rbase, RPT)], out_hbm.at[c, pl.ds(rbase, RPT)])

    NB = 2
    NIB = 4

    @functools.partial(
        pl.kernel,
        out_type=jax.ShapeDtypeStruct((NC, NP, H), jnp.float32),
        mesh=mesh,
        scratch_types=[
            pltpu.VMEM((NIB, 2, CH), jnp.int32),
            pltpu.VMEM((NB, CH, H), jnp.float32),
            pltpu.VMEM_SHARED((NP, H), jnp.float32),
            pltpu.SemaphoreType.DMA,
            pltpu.SemaphoreType.DMA,
            pltpu.SemaphoreType.DMA,
        ],
    )
    def _agg(g_hbm, edges_hbm, zeros_hbm, out_hbm,
             idx_v, rows_v, acc, semi, semg, sems):
        c = lax.axis_index("c")
        s = lax.axis_index("s")
        wid = s * NC + c
        kb = jnp.where(c == 0, K0, K1)
        rbase = s * RPT
        pltpu.sync_copy(zeros_hbm.at[pl.ds(rbase, RPT)], acc.at[pl.ds(rbase, RPT)])
        for j0 in range(NIB):
            pltpu.async_copy(edges_hbm.at[wid, j0], idx_v.at[j0], semi)
        plsc.subcore_barrier()
        for j0 in range(NB):
            pltpu.make_async_copy(edges_hbm.at[wid, j0], idx_v.at[j0], semi).wait()
            pltpu.async_copy(g_hbm.at[idx_v.at[j0, 0]], rows_v.at[j0], semg)

        def body(j, carry):
            b = j % NB
            i = j % NIB
            pltpu.make_async_copy(g_hbm.at[pl.ds(0, CH)], rows_v.at[b], semg).wait()

            pltpu.sync_copy(rows_v.at[b], acc.at[idx_v.at[i, 1]], add=True)

            @pl.when(j + NIB < kb)
            def _():
                pltpu.async_copy(edges_hbm.at[wid, j + NIB], idx_v.at[i], semi)

            @pl.when(j + NB < kb)
            def _():
                pltpu.make_async_copy(edges_hbm.at[wid, 0], idx_v.at[i], semi).wait()
                pltpu.async_copy(g_hbm.at[idx_v.at[(j + NB) % NIB, 0]], rows_v.at[b], semg)

            return carry

        lax.fori_loop(0, kb, body, 0)
        plsc.subcore_barrier()
        pltpu.sync_copy(acc.at[pl.ds(rbase, RPT)], out_hbm.at[c, pl.ds(rbase, RPT)])

    return _deg, _agg


R = 1000
_GRID = N // R


def _tc0_body(deg_ref, x_ref, w_ref, dinv_ref, g_ref):
    deg = deg_ref[0, :, 0:1] + deg_ref[1, :, 0:1]
    dinv = lax.rsqrt(deg)
    dinv_ref[...] = dinv
    h = jnp.dot(x_ref[...], w_ref[...], preferred_element_type=jnp.float32)
    g_ref[...] = h * dinv


def _tc0(degp, x, w0):
    return pl.pallas_call(
        _tc0_body,
        grid=(_GRID,),
        in_specs=[
            pl.BlockSpec((NC, R, LANES), lambda i: (0, i, 0)),
            pl.BlockSpec((R, D), lambda i: (i, 0)),
            pl.BlockSpec((D, H), lambda i: (0, 0)),
        ],
        out_specs=[
            pl.BlockSpec((R, 1), lambda i: (i, 0)),
            pl.BlockSpec((R, H), lambda i: (i, 0)),
        ],
        out_shape=[
            jax.ShapeDtypeStruct((N, 1), jnp.float32),
            jax.ShapeDtypeStruct((N, H), jnp.float32),
        ],
    )(degp, x, w0)


def _tcmid_body(sp_ref, dinv_ref, b_ref, w_ref, g_ref):
    dinv = dinv_ref[...]
    ssum = sp_ref[0] + sp_ref[1]
    y = jnp.maximum(ssum * dinv + b_ref[...], 0.0)
    g_ref[...] = jnp.dot(y, w_ref[...], preferred_element_type=jnp.float32) * dinv


def _tcmid(sp, dinv, b, w):
    return pl.pallas_call(
        _tcmid_body,
        grid=(_GRID,),
        in_specs=[
            pl.BlockSpec((NC, R, H), lambda i: (0, i, 0)),
            pl.BlockSpec((R, 1), lambda i: (i, 0)),
            pl.BlockSpec((1, H), lambda i: (0, 0)),
            pl.BlockSpec((H, H), lambda i: (0, 0)),
        ],
        out_specs=pl.BlockSpec((R, H), lambda i: (i, 0)),
        out_shape=jax.ShapeDtypeStruct((N, H), jnp.float32),
    )(sp, dinv, b, w)


def _tcfinal_body(sp_ref, dinv_ref, b2_ref, w1_ref, b1_ref, w2_ref, b2f_ref, w3_ref, b3_ref, out_ref):
    dinv = dinv_ref[...]
    y = jnp.maximum((sp_ref[0] + sp_ref[1]) * dinv + b2_ref[...], 0.0)
    z = jnp.maximum(jnp.dot(y, w1_ref[...], preferred_element_type=jnp.float32) + b1_ref[...], 0.0)
    z = jnp.maximum(jnp.dot(z, w2_ref[...], preferred_element_type=jnp.float32) + b2f_ref[...], 0.0)
    out_ref[...] = jnp.dot(z, w3_ref[...], preferred_element_type=jnp.float32) + b3_ref[...]


def _tcfinal(sp, dinv, b2, fc1W, fc1b, fc2W, fc2b, fc3Wp, fc3bp):
    return pl.pallas_call(
        _tcfinal_body,
        grid=(_GRID,),
        in_specs=[
            pl.BlockSpec((NC, R, H), lambda i: (0, i, 0)),
            pl.BlockSpec((R, 1), lambda i: (i, 0)),
            pl.BlockSpec((1, H), lambda i: (0, 0)),
            pl.BlockSpec((H, H), lambda i: (0, 0)),
            pl.BlockSpec((1, H), lambda i: (0, 0)),
            pl.BlockSpec((H, H), lambda i: (0, 0)),
            pl.BlockSpec((1, H), lambda i: (0, 0)),
            pl.BlockSpec((H, 128), lambda i: (0, 0)),
            pl.BlockSpec((1, 128), lambda i: (0, 0)),
        ],
        out_specs=pl.BlockSpec((R, 128), lambda i: (i, 0)),
        out_shape=jax.ShapeDtypeStruct((N, 128), jnp.float32),
    )(sp, dinv, b2, fc1W, fc1b, fc2W, fc2b, fc3Wp, fc3bp)


def kernel(x, edge_index, convW0, convb0, convW1, convb1, convW2, convb2,
           fc1W, fc1b, fc2W, fc2b, fc3W, fc3b, TRAIN=False):
    del TRAIN
    loop = jnp.arange(N, dtype=jnp.int32)
    pad = E_PAD - E_TOT

    def _layout(flat):
        e0 = flat[:NS * K0 * CH].reshape(NS, K0, CH)
        e1 = flat[NS * K0 * CH:].reshape(NS, K1, CH)
        e0 = jnp.pad(e0, ((0, 0), (0, KMAX - K0), (0, 0)))
        e1 = jnp.pad(e1, ((0, 0), (0, KMAX - K1), (0, 0)))
        return jnp.stack([e0, e1], axis=1).reshape(NW, KMAX, CH)

    srcp = _layout(jnp.concatenate([edge_index[0], loop, jnp.zeros((pad,), jnp.int32)]))
    dstp = _layout(jnp.concatenate([edge_index[1], loop, jnp.full((pad,), N, jnp.int32)]))
    edges = jnp.stack([srcp, dstp], axis=2)

    zeros_w = jnp.zeros((NP, H), jnp.float32)
    zeros_l = jnp.zeros((NP, LANES), jnp.float32)
    ones_l = jnp.ones((CH, LANES), jnp.float32)

    _deg, _agg = _sc_kernels()
    degp = _deg(edges, zeros_l, ones_l)
    dinv, g = _tc0(degp, x, convW0)
    for (b_prev, w_next) in ((convb0, convW1), (convb1, convW2)):
        sp = _agg(g, edges, zeros_w)
        g = _tcmid(sp, dinv, b_prev.reshape(1, H), w_next)
    sp = _agg(g, edges, zeros_w)

    fc3Wp = jnp.pad(fc3W, ((0, 0), (0, 128 - C)))
    fc3bp = jnp.pad(fc3b, (0, 128 - C)).reshape(1, 128)
    out = _tcfinal(sp, dinv, convb2.reshape(1, H), fc1W, fc1b.reshape(1, H),
                   fc2W, fc2b.reshape(1, H), fc3Wp, fc3bp)
    return out[:, :C]


# speedup vs baseline: 1.4284x; 1.1405x over previous
import functools

import jax
import jax.numpy as jnp
from jax import lax
from jax.experimental import pallas as pl
from jax.experimental.pallas import tpu as pltpu
from jax.experimental.pallas import tpu_sc as plsc

N = 10000
D = 128
H = 128
C = 121

NC = 2
NS = 16
NW = NC * NS
LANES = 16

NP = 10112
RPT = NP // NS
CH = 128
E_TOT = 320000 + N
NCHUNK = -(-E_TOT // (NW * CH))
E_PAD = NCHUNK * CH * NW
K0 = 97
K1 = 2 * NCHUNK - K0
KMAX = max(K0, K1)

@functools.cache
def _sc_kernels():
    mesh = plsc.VectorSubcoreMesh(core_axis_name="c", subcore_axis_name="s",
                                  num_cores=NC, num_subcores=NS)

    @functools.partial(
        pl.kernel,
        out_type=jax.ShapeDtypeStruct((NC, NP, LANES), jnp.float32),
        mesh=mesh,
        scratch_types=[
            pltpu.VMEM((KMAX, 2, CH), jnp.int32),
            pltpu.VMEM((CH, LANES), jnp.float32),
            pltpu.VMEM_SHARED((NP, LANES), jnp.float32),
            pltpu.SemaphoreType.DMA,
        ],
    )
    def _deg(edges_hbm, zeros_hbm, ones_hbm, out_hbm, idx_all, ones_v, acc, sem):
        c = lax.axis_index("c")
        s = lax.axis_index("s")
        wid = s * NC + c
        kb = jnp.where(c == 0, K0, K1)
        rbase = s * RPT
        pltpu.sync_copy(ones_hbm, ones_v)
        pltpu.sync_copy(edges_hbm.at[wid], idx_all)
        pltpu.sync_copy(zeros_hbm.at[pl.ds(rbase, RPT)], acc.at[pl.ds(rbase, RPT)])
        plsc.subcore_barrier()

        def body(j, carry):
            pltpu.sync_copy(ones_v, acc.at[idx_all.at[j, 1]], add=True)
            return carry

        lax.fori_loop(0, kb, body, 0)
        plsc.subcore_barrier()
        pltpu.sync_copy(acc.at[pl.ds(rbase, RPT)], out_hbm.at[c, pl.ds(rbase, RPT)])

    NB = 2
    NIB = 4

    @functools.partial(
        pl.kernel,
        out_type=jax.ShapeDtypeStruct((NC, NP, H), jnp.float32),
        mesh=mesh,
        scratch_types=[
            pltpu.VMEM((NIB, 2, CH), jnp.int32),
            pltpu.VMEM((NB, CH, H), jnp.float32),
            pltpu.VMEM_SHARED((NP, H), jnp.float32),
            pltpu.SemaphoreType.DMA,
            pltpu.SemaphoreType.DMA,
            pltpu.SemaphoreType.DMA,
        ],
    )
    def _agg(g_hbm, edges_hbm, zeros_hbm, out_hbm,
             idx_v, rows_v, acc, semi, semg, sems):
        c = lax.axis_index("c")
        s = lax.axis_index("s")
        wid = s * NC + c
        kb = jnp.where(c == 0, K0, K1)
        rbase = s * RPT
        pltpu.sync_copy(zeros_hbm.at[pl.ds(rbase, RPT)], acc.at[pl.ds(rbase, RPT)])
        for j0 in range(NIB):
            pltpu.async_copy(edges_hbm.at[wid, j0], idx_v.at[j0], semi)
        plsc.subcore_barrier()
        for j0 in range(NB):
            pltpu.make_async_copy(edges_hbm.at[wid, j0], idx_v.at[j0], semi).wait()
            pltpu.async_copy(g_hbm.at[idx_v.at[j0, 0]], rows_v.at[j0], semg)

        def body(j, carry):
            b = j % NB
            i = j % NIB
            pltpu.make_async_copy(g_hbm.at[pl.ds(0, CH)], rows_v.at[b], semg).wait()

            pltpu.sync_copy(rows_v.at[b], acc.at[idx_v.at[i, 1]], add=True)

            @pl.when(j + NIB < kb)
            def _():
                pltpu.async_copy(edges_hbm.at[wid, j + NIB], idx_v.at[i], semi)

            @pl.when(j + NB < kb)
            def _():
                pltpu.make_async_copy(edges_hbm.at[wid, 0], idx_v.at[i], semi).wait()
                pltpu.async_copy(g_hbm.at[idx_v.at[(j + NB) % NIB, 0]], rows_v.at[b], semg)

            return carry

        lax.fori_loop(0, kb, body, 0)
        plsc.subcore_barrier()
        pltpu.sync_copy(acc.at[pl.ds(rbase, RPT)], out_hbm.at[c, pl.ds(rbase, RPT)])

    return _deg, _agg


R = 1000
_GRID = N // R


def _tc0_body(deg_ref, x_ref, w_ref, dinv_ref, g_ref):
    deg = deg_ref[0, :, 0:1] + deg_ref[1, :, 0:1]
    dinv = lax.rsqrt(deg)
    dinv_ref[...] = dinv
    h = jnp.dot(x_ref[...], w_ref[...], preferred_element_type=jnp.float32)
    g_ref[...] = h * dinv


def _tc0(degp, x, w0):
    return pl.pallas_call(
        _tc0_body,
        grid=(_GRID,),
        in_specs=[
            pl.BlockSpec((NC, R, LANES), lambda i: (0, i, 0)),
            pl.BlockSpec((R, D), lambda i: (i, 0)),
            pl.BlockSpec((D, H), lambda i: (0, 0)),
        ],
        out_specs=[
            pl.BlockSpec((R, 1), lambda i: (i, 0)),
            pl.BlockSpec((R, H), lambda i: (i, 0)),
        ],
        out_shape=[
            jax.ShapeDtypeStruct((N, 1), jnp.float32),
            jax.ShapeDtypeStruct((N, H), jnp.float32),
        ],
    )(degp, x, w0)


def _tcmid_body(sp_ref, dinv_ref, b_ref, w_ref, g_ref):
    dinv = dinv_ref[...]
    ssum = sp_ref[0] + sp_ref[1]
    y = jnp.maximum(ssum * dinv + b_ref[...], 0.0)
    g_ref[...] = jnp.dot(y, w_ref[...], preferred_element_type=jnp.float32) * dinv


def _tcmid(sp, dinv, b, w):
    return pl.pallas_call(
        _tcmid_body,
        grid=(_GRID,),
        in_specs=[
            pl.BlockSpec((NC, R, H), lambda i: (0, i, 0)),
            pl.BlockSpec((R, 1), lambda i: (i, 0)),
            pl.BlockSpec((1, H), lambda i: (0, 0)),
            pl.BlockSpec((H, H), lambda i: (0, 0)),
        ],
        out_specs=pl.BlockSpec((R, H), lambda i: (i, 0)),
        out_shape=jax.ShapeDtypeStruct((N, H), jnp.float32),
    )(sp, dinv, b, w)


def _tcfinal_body(sp_ref, dinv_ref, b2_ref, w1_ref, b1_ref, w2_ref, b2f_ref, w3_ref, b3_ref, out_ref):
    dinv = dinv_ref[...]
    y = jnp.maximum((sp_ref[0] + sp_ref[1]) * dinv + b2_ref[...], 0.0)
    z = jnp.maximum(jnp.dot(y, w1_ref[...], preferred_element_type=jnp.float32) + b1_ref[...], 0.0)
    z = jnp.maximum(jnp.dot(z, w2_ref[...], preferred_element_type=jnp.float32) + b2f_ref[...], 0.0)
    out_ref[...] = jnp.dot(z, w3_ref[...], preferred_element_type=jnp.float32) + b3_ref[...]


def _tcfinal(sp, dinv, b2, fc1W, fc1b, fc2W, fc2b, fc3Wp, fc3bp):
    return pl.pallas_call(
        _tcfinal_body,
        grid=(_GRID,),
        in_specs=[
            pl.BlockSpec((NC, R, H), lambda i: (0, i, 0)),
            pl.BlockSpec((R, 1), lambda i: (i, 0)),
            pl.BlockSpec((1, H), lambda i: (0, 0)),
            pl.BlockSpec((H, H), lambda i: (0, 0)),
            pl.BlockSpec((1, H), lambda i: (0, 0)),
            pl.BlockSpec((H, H), lambda i: (0, 0)),
            pl.BlockSpec((1, H), lambda i: (0, 0)),
            pl.BlockSpec((H, 128), lambda i: (0, 0)),
            pl.BlockSpec((1, 128), lambda i: (0, 0)),
        ],
        out_specs=pl.BlockSpec((R, 128), lambda i: (i, 0)),
        out_shape=jax.ShapeDtypeStruct((N, 128), jnp.float32),
    )(sp, dinv, b2, fc1W, fc1b, fc2W, fc2b, fc3Wp, fc3bp)


def kernel(x, edge_index, convW0, convb0, convW1, convb1, convW2, convb2,
           fc1W, fc1b, fc2W, fc2b, fc3W, fc3b, TRAIN=False):
    del TRAIN
    loop = jnp.arange(N, dtype=jnp.int32)
    pad = E_PAD - E_TOT

    def _layout(flat):
        e0 = flat[:NS * K0 * CH].reshape(NS, K0, CH)
        e1 = flat[NS * K0 * CH:].reshape(NS, K1, CH)
        e0 = jnp.pad(e0, ((0, 0), (0, KMAX - K0), (0, 0)))
        e1 = jnp.pad(e1, ((0, 0), (0, KMAX - K1), (0, 0)))
        return jnp.stack([e0, e1], axis=1).reshape(NW, KMAX, CH)

    srcp = _layout(jnp.concatenate([edge_index[0], loop, jnp.zeros((pad,), jnp.int32)]))
    dstp = _layout(jnp.concatenate([edge_index[1], loop, jnp.full((pad,), N, jnp.int32)]))
    edges = jnp.stack([srcp, dstp], axis=2)

    zeros_w = jnp.zeros((NP, H), jnp.float32)
    zeros_l = jnp.zeros((NP, LANES), jnp.float32)
    ones_l = jnp.ones((CH, LANES), jnp.float32)

    _deg, _agg = _sc_kernels()
    degp = _deg(edges, zeros_l, ones_l)
    dinv, g = _tc0(degp, x, convW0)
    for (b_prev, w_next) in ((convb0, convW1), (convb1, convW2)):
        sp = _agg(g, edges, zeros_w)
        g = _tcmid(sp, dinv, b_prev.reshape(1, H), w_next)
    sp = _agg(g, edges, zeros_w)

    fc3Wp = jnp.pad(fc3W, ((0, 0), (0, 128 - C)))
    fc3bp = jnp.pad(fc3b, (0, 128 - C)).reshape(1, 128)
    out = _tcfinal(sp, dinv, convb2.reshape(1, H), fc1W, fc1b.reshape(1, H),
                   fc2W, fc2b.reshape(1, H), fc3Wp, fc3bp)
    return out[:, :C]


# speedup vs baseline: 1.4578x; 1.0206x over previous
import functools

import jax
import jax.numpy as jnp
from jax import lax
from jax.experimental import pallas as pl
from jax.experimental.pallas import tpu as pltpu
from jax.experimental.pallas import tpu_sc as plsc

N = 10000
D = 128
H = 128
C = 121

NC = 2
NS = 16
NW = NC * NS
LANES = 16

NP = 10112
RPT = NP // NS
CH = 128
E_TOT = 320000 + N
NCHUNK = -(-E_TOT // (NW * CH))
E_PAD = NCHUNK * CH * NW
K0 = 110
K1 = 2 * NCHUNK - K0
KMAX = max(K0, K1)

@functools.cache
def _sc_kernels():
    mesh = plsc.VectorSubcoreMesh(core_axis_name="c", subcore_axis_name="s",
                                  num_cores=NC, num_subcores=NS)

    @functools.partial(
        pl.kernel,
        out_type=jax.ShapeDtypeStruct((NC, NP, LANES), jnp.float32),
        mesh=mesh,
        scratch_types=[
            pltpu.VMEM((KMAX, 2, CH), jnp.int32),
            pltpu.VMEM((CH, LANES), jnp.float32),
            pltpu.VMEM_SHARED((NP, LANES), jnp.float32),
            pltpu.SemaphoreType.DMA,
        ],
    )
    def _deg(edges_hbm, zeros_hbm, ones_hbm, out_hbm, idx_all, ones_v, acc, sem):
        c = lax.axis_index("c")
        s = lax.axis_index("s")
        wid = s * NC + c
        kb = jnp.where(c == 0, K0, K1)
        rbase = s * RPT
        pltpu.sync_copy(ones_hbm, ones_v)
        pltpu.sync_copy(edges_hbm.at[wid], idx_all)
        pltpu.sync_copy(zeros_hbm.at[pl.ds(rbase, RPT)], acc.at[pl.ds(rbase, RPT)])
        plsc.subcore_barrier()

        def body(j, carry):
            pltpu.sync_copy(ones_v, acc.at[idx_all.at[j, 1]], add=True)
            return carry

        lax.fori_loop(0, kb, body, 0)
        plsc.subcore_barrier()
        pltpu.sync_copy(acc.at[pl.ds(rbase, RPT)], out_hbm.at[c, pl.ds(rbase, RPT)])

    NB = 2
    NIB = 4

    @functools.partial(
        pl.kernel,
        out_type=jax.ShapeDtypeStruct((NC, NP, H), jnp.float32),
        mesh=mesh,
        scratch_types=[
            pltpu.VMEM((NIB, 2, CH), jnp.int32),
            pltpu.VMEM((NB, CH, H), jnp.float32),
            pltpu.VMEM_SHARED((NP, H), jnp.float32),
            pltpu.SemaphoreType.DMA,
            pltpu.SemaphoreType.DMA,
            pltpu.SemaphoreType.DMA,
        ],
    )
    def _agg(g_hbm, edges_hbm, zeros_hbm, out_hbm,
             idx_v, rows_v, acc, semi, semg, sems):
        c = lax.axis_index("c")
        s = lax.axis_index("s")
        wid = s * NC + c
        kb = jnp.where(c == 0, K0, K1)
        rbase = s * RPT
        pltpu.sync_copy(zeros_hbm.at[pl.ds(rbase, RPT)], acc.at[pl.ds(rbase, RPT)])
        for j0 in range(NIB):
            pltpu.async_copy(edges_hbm.at[wid, j0], idx_v.at[j0], semi)
        plsc.subcore_barrier()
        for j0 in range(NB):
            pltpu.make_async_copy(edges_hbm.at[wid, j0], idx_v.at[j0], semi).wait()
            pltpu.async_copy(g_hbm.at[idx_v.at[j0, 0]], rows_v.at[j0], semg)

        def body(j, carry):
            b = j % NB
            i = j % NIB
            pltpu.make_async_copy(g_hbm.at[pl.ds(0, CH)], rows_v.at[b], semg).wait()

            pltpu.sync_copy(rows_v.at[b], acc.at[idx_v.at[i, 1]], add=True)

            @pl.when(j + NIB < kb)
            def _():
                pltpu.async_copy(edges_hbm.at[wid, j + NIB], idx_v.at[i], semi)

            @pl.when(j + NB < kb)
            def _():
                pltpu.make_async_copy(edges_hbm.at[wid, 0], idx_v.at[i], semi).wait()
                pltpu.async_copy(g_hbm.at[idx_v.at[(j + NB) % NIB, 0]], rows_v.at[b], semg)

            return carry

        lax.fori_loop(0, kb, body, 0)
        plsc.subcore_barrier()
        pltpu.sync_copy(acc.at[pl.ds(rbase, RPT)], out_hbm.at[c, pl.ds(rbase, RPT)])

    return _deg, _agg


R = 1000
_GRID = N // R


def _tc0_body(deg_ref, x_ref, w_ref, dinv_ref, g_ref):
    deg = deg_ref[0, :, 0:1] + deg_ref[1, :, 0:1]
    dinv = lax.rsqrt(deg)
    dinv_ref[...] = dinv
    h = jnp.dot(x_ref[...], w_ref[...], preferred_element_type=jnp.float32)
    g_ref[...] = h * dinv


def _tc0(degp, x, w0):
    return pl.pallas_call(
        _tc0_body,
        grid=(_GRID,),
        in_specs=[
            pl.BlockSpec((NC, R, LANES), lambda i: (0, i, 0)),
            pl.BlockSpec((R, D), lambda i: (i, 0)),
            pl.BlockSpec((D, H), lambda i: (0, 0)),
        ],
        out_specs=[
            pl.BlockSpec((R, 1), lambda i: (i, 0)),
            pl.BlockSpec((R, H), lambda i: (i, 0)),
        ],
        out_shape=[
            jax.ShapeDtypeStruct((N, 1), jnp.float32),
            jax.ShapeDtypeStruct((N, H), jnp.float32),
        ],
    )(degp, x, w0)


def _tcmid_body(sp_ref, dinv_ref, b_ref, w_ref, g_ref):
    dinv = dinv_ref[...]
    ssum = sp_ref[0] + sp_ref[1]
    y = jnp.maximum(ssum * dinv + b_ref[...], 0.0)
    g_ref[...] = jnp.dot(y, w_ref[...], preferred_element_type=jnp.float32) * dinv


def _tcmid(sp, dinv, b, w):
    return pl.pallas_call(
        _tcmid_body,
        grid=(_GRID,),
        in_specs=[
            pl.BlockSpec((NC, R, H), lambda i: (0, i, 0)),
            pl.BlockSpec((R, 1), lambda i: (i, 0)),
            pl.BlockSpec((1, H), lambda i: (0, 0)),
            pl.BlockSpec((H, H), lambda i: (0, 0)),
        ],
        out_specs=pl.BlockSpec((R, H), lambda i: (i, 0)),
        out_shape=jax.ShapeDtypeStruct((N, H), jnp.float32),
    )(sp, dinv, b, w)


def _tcfinal_body(sp_ref, dinv_ref, b2_ref, w1_ref, b1_ref, w2_ref, b2f_ref, w3_ref, b3_ref, out_ref):
    dinv = dinv_ref[...]
    y = jnp.maximum((sp_ref[0] + sp_ref[1]) * dinv + b2_ref[...], 0.0)
    z = jnp.maximum(jnp.dot(y, w1_ref[...], preferred_element_type=jnp.float32) + b1_ref[...], 0.0)
    z = jnp.maximum(jnp.dot(z, w2_ref[...], preferred_element_type=jnp.float32) + b2f_ref[...], 0.0)
    out_ref[...] = jnp.dot(z, w3_ref[...], preferred_element_type=jnp.float32) + b3_ref[...]


def _tcfinal(sp, dinv, b2, fc1W, fc1b, fc2W, fc2b, fc3Wp, fc3bp):
    return pl.pallas_call(
        _tcfinal_body,
        grid=(_GRID,),
        in_specs=[
            pl.BlockSpec((NC, R, H), lambda i: (0, i, 0)),
            pl.BlockSpec((R, 1), lambda i: (i, 0)),
            pl.BlockSpec((1, H), lambda i: (0, 0)),
            pl.BlockSpec((H, H), lambda i: (0, 0)),
            pl.BlockSpec((1, H), lambda i: (0, 0)),
            pl.BlockSpec((H, H), lambda i: (0, 0)),
            pl.BlockSpec((1, H), lambda i: (0, 0)),
            pl.BlockSpec((H, 128), lambda i: (0, 0)),
            pl.BlockSpec((1, 128), lambda i: (0, 0)),
        ],
        out_specs=pl.BlockSpec((R, 128), lambda i: (i, 0)),
        out_shape=jax.ShapeDtypeStruct((N, 128), jnp.float32),
    )(sp, dinv, b2, fc1W, fc1b, fc2W, fc2b, fc3Wp, fc3bp)


def kernel(x, edge_index, convW0, convb0, convW1, convb1, convW2, convb2,
           fc1W, fc1b, fc2W, fc2b, fc3W, fc3b, TRAIN=False):
    del TRAIN
    loop = jnp.arange(N, dtype=jnp.int32)
    pad = E_PAD - E_TOT

    def _layout(flat):
        e0 = flat[:NS * K0 * CH].reshape(NS, K0, CH)
        e1 = flat[NS * K0 * CH:].reshape(NS, K1, CH)
        e0 = jnp.pad(e0, ((0, 0), (0, KMAX - K0), (0, 0)))
        e1 = jnp.pad(e1, ((0, 0), (0, KMAX - K1), (0, 0)))
        return jnp.stack([e0, e1], axis=1).reshape(NW, KMAX, CH)

    srcp = _layout(jnp.concatenate([edge_index[0], loop, jnp.zeros((pad,), jnp.int32)]))
    dstp = _layout(jnp.concatenate([edge_index[1], loop, jnp.full((pad,), N, jnp.int32)]))
    edges = jnp.stack([srcp, dstp], axis=2)

    zeros_w = jnp.zeros((NP, H), jnp.float32)
    zeros_l = jnp.zeros((NP, LANES), jnp.float32)
    ones_l = jnp.ones((CH, LANES), jnp.float32)

    _deg, _agg = _sc_kernels()
    degp = _deg(edges, zeros_l, ones_l)
    dinv, g = _tc0(degp, x, convW0)
    for (b_prev, w_next) in ((convb0, convW1), (convb1, convW2)):
        sp = _agg(g, edges, zeros_w)
        g = _tcmid(sp, dinv, b_prev.reshape(1, H), w_next)
    sp = _agg(g, edges, zeros_w)

    fc3Wp = jnp.pad(fc3W, ((0, 0), (0, 128 - C)))
    fc3bp = jnp.pad(fc3b, (0, 128 - C)).reshape(1, 128)
    out = _tcfinal(sp, dinv, convb2.reshape(1, H), fc1W, fc1b.reshape(1, H),
                   fc2W, fc2b.reshape(1, H), fc3Wp, fc3bp)
    return out[:, :C]
